# baseline (speedup 1.0000x reference)
import functools

import jax
import jax.numpy as jnp
from jax import lax
from jax.experimental import pallas as pl
from jax.experimental.pallas import tpu as pltpu
from jax.experimental.pallas import tpu_sc as plsc

N_POS = 16384
N_NEG = 81920
EMB = 64
CHUNK = 128

_info = plsc.get_sparse_core_info()
NC, NS, LANES = _info.num_cores, _info.num_subcores, _info.num_lanes
NW = NC * NS
POS_PER_W = N_POS // NW
NEG_PER_W = N_NEG // NW


def _log_sigmoid(x):
    ax = jnp.abs(x)
    u = jnp.exp(-ax)
    s = u / (u + 2.0)
    s2 = s * s
    p = 2.0 * s * (1.0 + s2 * (1.0 / 3.0 + s2 * (1.0 / 5.0 + s2 * (1.0 / 7.0 + s2 * (1.0 / 9.0)))))
    return jnp.minimum(x, 0.0) - p


def _chunk_scores(wrows, vrows, sign):
    lane = lax.iota(jnp.int32, LANES)
    total = jnp.zeros((LANES,), jnp.float32)
    for g in range(CHUNK // LANES):
        rows = lane + (g * LANES)
        scores = jnp.zeros((LANES,), jnp.float32)
        for k in range(EMB):
            col = jnp.full((LANES,), k, jnp.int32)
            wv = plsc.load_gather(wrows, [rows, col])
            vv = plsc.load_gather(vrows, [rows, col])
            scores = scores + wv * vv
        if sign < 0:
            scores = -scores
        total = total + _log_sigmoid(scores)
    return total


def _make_kernel():
    mesh = plsc.VectorSubcoreMesh(core_axis_name="c", subcore_axis_name="s")

    @functools.partial(
        pl.kernel,
        mesh=mesh,
        out_type=jax.ShapeDtypeStruct((NW, LANES), jnp.float32),
        compiler_params=pltpu.CompilerParams(
            needs_layout_passes=False, use_tc_tiling_on_sc=False),
        scratch_types=[
            pltpu.VMEM((CHUNK,), jnp.int32),
            pltpu.VMEM((CHUNK,), jnp.int32),
            pltpu.VMEM((CHUNK, EMB), jnp.float32),
            pltpu.VMEM((CHUNK, EMB), jnp.float32),
            pltpu.VMEM((LANES,), jnp.float32),
            pltpu.SemaphoreType.DMA,
            pltpu.SemaphoreType.DMA,
        ],
    )
    def skipgram(pos_w, pos_v, neg_w, neg_v, W, V, out,
                 wi_v, vi_v, wrows, vrows, acc_v, semw, semv):
        wid = lax.axis_index("s") * NC + lax.axis_index("c")

        def run_chunk(w_idx_hbm, v_idx_hbm, start, sign, acc):
            pltpu.sync_copy(w_idx_hbm.at[pl.ds(start, CHUNK)], wi_v)
            pltpu.sync_copy(v_idx_hbm.at[pl.ds(start, CHUNK)], vi_v)
            cw = pltpu.async_copy(W.at[wi_v], wrows, semw)
            cv = pltpu.async_copy(V.at[vi_v], vrows, semv)
            cw.wait()
            cv.wait()
            return acc + _chunk_scores(wrows, vrows, sign)

        pos_base = wid * POS_PER_W
        neg_base = wid * NEG_PER_W

        def pos_body(c, acc):
            return run_chunk(pos_w, pos_v, pos_base + c * CHUNK, -1, acc)

        def neg_body(c, acc):
            return run_chunk(neg_w, neg_v, neg_base + c * CHUNK, 1, acc)

        acc = jnp.zeros((LANES,), jnp.float32)
        acc = lax.fori_loop(0, POS_PER_W // CHUNK, pos_body, acc)
        acc = lax.fori_loop(0, NEG_PER_W // CHUNK, neg_body, acc)
        acc_v[...] = acc
        pltpu.sync_copy(acc_v, out.at[wid])

    return skipgram


_skipgram_kernel = _make_kernel()


def kernel(pos_w, pos_v, neg_w, neg_v, W, V):
    partials = _skipgram_kernel(pos_w, pos_v, neg_w, neg_v, W, V)
    return -jnp.sum(partials)
